# baseline (speedup 1.0000x reference)
import jax
import jax.numpy as jnp
from jax import lax
from jax.experimental import pallas as pl
from jax.experimental.pallas import tpu as pltpu
from jax.experimental.pallas import tpu_sc as plsc

_DIM = 64
_BASE = 10000.0

_NC = 2
_NS = 16
_NW = _NC * _NS

_B = 4096
_L = 200
_BG = _B // _NW
_NPAIR = _L // 2

_V = 1000000
_NT = (_V + 127) // 128
_VPAD = _NT * 128
_T2ROWS = _VPAD // 2
_NT_LO = _NT // _NW
_NT_EXTRA = _NT - _NT_LO * _NW


def _make_pe():
    pos = jnp.arange(_L, dtype=jnp.float32)[:, None]
    div = jnp.exp(
        jnp.arange(0, _DIM, 2, dtype=jnp.float32) * (-jnp.log(_BASE) / _DIM)
    )
    pe = jnp.zeros((_L, _DIM), dtype=jnp.float32)
    pe = pe.at[:, 0::2].set(jnp.sin(pos * div))
    pe = pe.at[:, 1::2].set(jnp.cos(pos * div))
    return pe


def _relayout_body(tabt_hbm, tail2_hbm, t2_hbm, sbufa, sbufb, obufa, obufb,
                   gsema, gsemb, osema, osemb):
    c = lax.axis_index("c")
    s = lax.axis_index("s")
    wid = s * _NC + c

    lo = jnp.minimum(wid, _NT_EXTRA)
    start = wid * _NT_LO + lo
    cnt = jnp.where(wid < _NT_EXTRA, _NT_LO + 1, _NT_LO)
    cnt = jnp.where(wid == _NW - 1, cnt - 1, cnt)

    fvec = [lax.iota(jnp.int32, 16) + 16 * k for k in range(4)]

    def fetch(v, buf, sem):
        return pltpu.async_copy(tabt_hbm.at[:, pl.ds(v * 128, 128)], buf, sem)

    def transpose(sbuf, obuf):
        def rows(q, _):
            u0 = 2 * q
            for d in range(2):
                uv = lax.full((16,), 0, jnp.int32) + (u0 + d)
                for k in range(4):
                    col = plsc.load_gather(sbuf, [fvec[k], uv])
                    obuf[q, pl.ds(d * 64 + k * 16, 16)] = col
            return 0

        lax.fori_loop(0, 64, rows, 0)

    def put(v, obuf, sem):
        return pltpu.async_copy(t2_hbm.at[pl.ds(v * 64, 64), :], obuf, sem)

    def _serial_step(i, _):
        va = start + i
        pltpu.sync_copy(tabt_hbm.at[:, pl.ds(va * 128, 128)], sbufa)
        transpose(sbufa, obufa)
        pltpu.sync_copy(obufa, t2_hbm.at[pl.ds(va * 64, 64), :])
        return 0

    lax.fori_loop(0, cnt, _serial_step, 0)

    @pl.when(cnt < 0)
    def _():
        fetch(start, sbufa, gsema)

        def step(i, _):
            va = start + 2 * i
            in_b = va + 1 < start + cnt

            @pl.when(in_b)
            def _():
                fetch(va + 1, sbufb, gsemb)

            pltpu.make_async_copy(
                tabt_hbm.at[:, pl.ds(va * 128, 128)], sbufa, gsema
            ).wait()

            @pl.when(i > 0)
            def _():
                pltpu.make_async_copy(
                    obufa, t2_hbm.at[pl.ds(va * 64, 64), :], osema
                ).wait()

            transpose(sbufa, obufa)
            put(va, obufa, osema)

            @pl.when(va + 2 < start + cnt)
            def _():
                fetch(va + 2, sbufa, gsema)

            @pl.when(in_b)
            def _():
                pltpu.make_async_copy(
                    tabt_hbm.at[:, pl.ds((va + 1) * 128, 128)], sbufb, gsemb
                ).wait()

                @pl.when(i > 0)
                def _():
                    pltpu.make_async_copy(
                        obufb, t2_hbm.at[pl.ds(va * 64, 64), :], osemb
                    ).wait()

                transpose(sbufb, obufb)
                put(va + 1, obufb, osemb)

            return 0

        nsteps = lax.div(cnt + 1, 2)
        lax.fori_loop(0, nsteps, step, 0)

        pltpu.make_async_copy(
            obufa, t2_hbm.at[pl.ds(start * 64, 64), :], osema
        ).wait()

        @pl.when(cnt > 1)
        def _():
            pltpu.make_async_copy(
                obufb, t2_hbm.at[pl.ds(start * 64, 64), :], osemb
            ).wait()

    @pl.when(wid == _NW - 1)
    def _():
        pltpu.sync_copy(tail2_hbm, sbufa.at[pl.ds(0, 32), :])
        pltpu.sync_copy(
            sbufa.at[pl.ds(0, 32), :],
            t2_hbm.at[pl.ds((_NT - 1) * 64, 32), :],
        )


_TAIL0 = (_NT - 1) * 128


def _gather_body(
    xt_hbm, tab_hbm, pe_hbm, out_hbm,
    idx_v, pe_v, gbufa, gbufb, obufa, obufb,
    gsema, gsemb, osema, osemb,
):
    c = lax.axis_index("c")
    s = lax.axis_index("s")
    wid = s * _NC + c

    pltpu.sync_copy(xt_hbm.at[:, pl.ds(wid * _BG, _BG)], idx_v)
    pltpu.sync_copy(pe_hbm, pe_v)

    row_ids = [lax.iota(jnp.int32, 16) + l * 16 for l in range(_BG // 16)]

    def gather(j, buf, sem):
        return pltpu.async_copy(tab_hbm.at[idx_v.at[j]], buf, sem)

    def transpose_add(j, gbuf, obuf):
        pe_row = [pe_v[j, pl.ds(k * 16, 16)] for k in range(_DIM // 16)]
        for f in range(_DIM):
            pe_f = pe_row[f // 16][f % 16]
            fv = lax.full((16,), 0, jnp.int32) + f
            for l in range(_BG // 16):
                col = plsc.load_gather(gbuf, [row_ids[l], fv])
                obuf[f // 8, f % 8, pl.ds(l * 16, 16)] = col + pe_f

    def put(j, obuf, sem):
        return pltpu.async_copy(obuf, out_hbm.at[j, :, wid, :, :], sem)

    gather(0, gbufa, gsema)

    def pair(m, _):
        ja = 2 * m
        gather(ja + 1, gbufb, gsemb)
        pltpu.make_async_copy(tab_hbm.at[idx_v.at[ja]], gbufa, gsema).wait()
        transpose_add(ja, gbufa, obufa)

        @pl.when(m > 0)
        def _():
            pltpu.make_async_copy(
                obufb, out_hbm.at[ja - 1, :, wid, :, :], osemb
            ).wait()

        out_a = put(ja, obufa, osema)

        @pl.when(m < _NPAIR - 1)
        def _():
            gather(ja + 2, gbufa, gsema)

        pltpu.make_async_copy(
            tab_hbm.at[idx_v.at[ja + 1]], gbufb, gsemb
        ).wait()
        transpose_add(ja + 1, gbufb, obufb)
        out_a.wait()
        put(ja + 1, obufb, osemb)
        return 0

    lax.fori_loop(0, _NPAIR, pair, 0)
    pltpu.make_async_copy(
        obufb, out_hbm.at[_L - 1, :, wid, :, :], osemb
    ).wait()


@jax.jit
def kernel(x, table):
    pe = _make_pe()
    xt = x.T.astype(jnp.int32)
    tabt = table.T

    mesh = plsc.VectorSubcoreMesh(core_axis_name="c", subcore_axis_name="s")

    t2 = pl.kernel(
        _relayout_body,
        out_type=jax.ShapeDtypeStruct((_T2ROWS, 128), jnp.float32),
        mesh=mesh,
        scratch_types=[
            pltpu.VMEM((_DIM, 128), jnp.float32),
            pltpu.VMEM((_DIM, 128), jnp.float32),
            pltpu.VMEM((_DIM, 128), jnp.float32),
            pltpu.VMEM((_DIM, 128), jnp.float32),
            pltpu.SemaphoreType.DMA,
            pltpu.SemaphoreType.DMA,
            pltpu.SemaphoreType.DMA,
            pltpu.SemaphoreType.DMA,
        ],
        compiler_params=pltpu.CompilerParams(
            use_tc_tiling_on_sc=True, needs_layout_passes=False
        ),
    )(tabt, table[_TAIL0:, :].reshape(32, 128))

    tab_lin = t2.reshape(_VPAD, _DIM)

    out5 = pl.kernel(
        _gather_body,
        out_type=jax.ShapeDtypeStruct((_L, 8, _NW, 8, _BG), jnp.float32),
        mesh=mesh,
        scratch_types=[
            pltpu.VMEM((_L, _BG), jnp.int32),
            pltpu.VMEM((_L, _DIM), jnp.float32),
            pltpu.VMEM((_BG, _DIM), jnp.float32),
            pltpu.VMEM((_BG, _DIM), jnp.float32),
            pltpu.VMEM((8, 8, _BG), jnp.float32),
            pltpu.VMEM((8, 8, _BG), jnp.float32),
            pltpu.SemaphoreType.DMA,
            pltpu.SemaphoreType.DMA,
            pltpu.SemaphoreType.DMA,
            pltpu.SemaphoreType.DMA,
        ],
        compiler_params=pltpu.CompilerParams(
            use_tc_tiling_on_sc=False, needs_layout_passes=False
        ),
    )(xt, tab_lin, pe)
    return out5.transpose(2, 4, 0, 1, 3).reshape(_B, _L, _DIM)


# speedup vs baseline: 1.0854x; 1.0854x over previous
import jax
import jax.numpy as jnp
from jax import lax
from jax.experimental import pallas as pl
from jax.experimental.pallas import tpu as pltpu
from jax.experimental.pallas import tpu_sc as plsc

_DIM = 64
_BASE = 10000.0

_NC = 2
_NS = 16
_NW = _NC * _NS

_B = 4096
_L = 200
_BG = _B // _NW
_NPAIR = _L // 2

_V = 1000000
_NT = (_V + 127) // 128
_VPAD = _NT * 128
_T2ROWS = _VPAD // 2
_NT_LO = _NT // _NW
_NT_EXTRA = _NT - _NT_LO * _NW


def _make_pe():
    pos = jnp.arange(_L, dtype=jnp.float32)[:, None]
    div = jnp.exp(
        jnp.arange(0, _DIM, 2, dtype=jnp.float32) * (-jnp.log(_BASE) / _DIM)
    )
    pe = jnp.zeros((_L, _DIM), dtype=jnp.float32)
    pe = pe.at[:, 0::2].set(jnp.sin(pos * div))
    pe = pe.at[:, 1::2].set(jnp.cos(pos * div))
    return pe


def _relayout_body(tabt_hbm, tail2_hbm, t2_hbm, sbufa, sbufb, obufa, obufb,
                   gsema, gsemb, osema, osemb):
    c = lax.axis_index("c")
    s = lax.axis_index("s")
    wid = s * _NC + c

    lo = jnp.minimum(wid, _NT_EXTRA)
    start = wid * _NT_LO + lo
    cnt = jnp.where(wid < _NT_EXTRA, _NT_LO + 1, _NT_LO)
    cnt = jnp.where(wid == _NW - 1, cnt - 1, cnt)

    fvec = [lax.iota(jnp.int32, 16) + 16 * k for k in range(4)]

    def fetch(v, buf, sem):
        return pltpu.async_copy(tabt_hbm.at[:, pl.ds(v * 128, 128)], buf, sem)

    def transpose(sbuf, obuf):
        def rows(q, _):
            u0 = 2 * q
            for d in range(2):
                uv = lax.full((16,), 0, jnp.int32) + (u0 + d)
                for k in range(4):
                    col = plsc.load_gather(sbuf, [fvec[k], uv])
                    obuf[q, pl.ds(d * 64 + k * 16, 16)] = col
            return 0

        lax.fori_loop(0, 64, rows, 0)

    def put(v, obuf, sem):
        return pltpu.async_copy(t2_hbm.at[pl.ds(v * 64, 64), :], obuf, sem)

    fetch(start, sbufa, gsema)

    def step(i, _):
        va = start + 2 * i
        in_b = va + 1 < start + cnt

        @pl.when(in_b)
        def _():
            fetch(va + 1, sbufb, gsemb)

        pltpu.make_async_copy(
            tabt_hbm.at[:, pl.ds(va * 128, 128)], sbufa, gsema
        ).wait()
        transpose(sbufa, obufa)
        pltpu.sync_copy(obufa, t2_hbm.at[pl.ds(va * 64, 64), :])

        @pl.when(va + 2 < start + cnt)
        def _():
            fetch(va + 2, sbufa, gsema)

        @pl.when(in_b)
        def _():
            pltpu.make_async_copy(
                tabt_hbm.at[:, pl.ds((va + 1) * 128, 128)], sbufb, gsemb
            ).wait()
            transpose(sbufb, obufb)
            pltpu.sync_copy(obufb, t2_hbm.at[pl.ds((va + 1) * 64, 64), :])

        return 0

    nsteps = lax.div(cnt + 1, 2)
    lax.fori_loop(0, nsteps, step, 0)

    @pl.when(wid == _NW - 1)
    def _():
        pltpu.sync_copy(tail2_hbm, sbufa.at[pl.ds(0, 32), :])
        pltpu.sync_copy(
            sbufa.at[pl.ds(0, 32), :],
            t2_hbm.at[pl.ds((_NT - 1) * 64, 32), :],
        )


_TAIL0 = (_NT - 1) * 128


def _gather_body(
    xt_hbm, tab_hbm, pe_hbm, out_hbm,
    idx_v, pe_v, gbufa, gbufb, obufa, obufb,
    gsema, gsemb, osema, osemb,
):
    c = lax.axis_index("c")
    s = lax.axis_index("s")
    wid = s * _NC + c

    pltpu.sync_copy(xt_hbm.at[:, pl.ds(wid * _BG, _BG)], idx_v)
    pltpu.sync_copy(pe_hbm, pe_v)

    row_ids = [lax.iota(jnp.int32, 16) + l * 16 for l in range(_BG // 16)]

    def gather(j, buf, sem):
        return pltpu.async_copy(tab_hbm.at[idx_v.at[j]], buf, sem)

    def transpose_add(j, gbuf, obuf):
        pe_row = [pe_v[j, pl.ds(k * 16, 16)] for k in range(_DIM // 16)]
        for f in range(_DIM):
            pe_f = pe_row[f // 16][f % 16]
            fv = lax.full((16,), 0, jnp.int32) + f
            for l in range(_BG // 16):
                col = plsc.load_gather(gbuf, [row_ids[l], fv])
                obuf[f // 8, f % 8, pl.ds(l * 16, 16)] = col + pe_f

    def put(j, obuf, sem):
        return pltpu.async_copy(obuf, out_hbm.at[j, :, wid, :, :], sem)

    gather(0, gbufa, gsema)

    def pair(m, _):
        ja = 2 * m
        gather(ja + 1, gbufb, gsemb)
        pltpu.make_async_copy(tab_hbm.at[idx_v.at[ja]], gbufa, gsema).wait()
        transpose_add(ja, gbufa, obufa)

        @pl.when(m > 0)
        def _():
            pltpu.make_async_copy(
                obufb, out_hbm.at[ja - 1, :, wid, :, :], osemb
            ).wait()

        out_a = put(ja, obufa, osema)

        @pl.when(m < _NPAIR - 1)
        def _():
            gather(ja + 2, gbufa, gsema)

        pltpu.make_async_copy(
            tab_hbm.at[idx_v.at[ja + 1]], gbufb, gsemb
        ).wait()
        transpose_add(ja + 1, gbufb, obufb)
        out_a.wait()
        put(ja + 1, obufb, osemb)
        return 0

    lax.fori_loop(0, _NPAIR, pair, 0)
    pltpu.make_async_copy(
        obufb, out_hbm.at[_L - 1, :, wid, :, :], osemb
    ).wait()


@jax.jit
def kernel(x, table):
    pe = _make_pe()
    xt = x.T.astype(jnp.int32)
    tabt = table.T

    mesh = plsc.VectorSubcoreMesh(core_axis_name="c", subcore_axis_name="s")

    t2 = pl.kernel(
        _relayout_body,
        out_type=jax.ShapeDtypeStruct((_T2ROWS, 128), jnp.float32),
        mesh=mesh,
        scratch_types=[
            pltpu.VMEM((_DIM, 128), jnp.float32),
            pltpu.VMEM((_DIM, 128), jnp.float32),
            pltpu.VMEM((_DIM, 128), jnp.float32),
            pltpu.VMEM((_DIM, 128), jnp.float32),
            pltpu.SemaphoreType.DMA,
            pltpu.SemaphoreType.DMA,
            pltpu.SemaphoreType.DMA,
            pltpu.SemaphoreType.DMA,
        ],
        compiler_params=pltpu.CompilerParams(
            use_tc_tiling_on_sc=True, needs_layout_passes=False
        ),
    )(tabt, table[_TAIL0:, :].reshape(32, 128))

    tab_lin = t2.reshape(_VPAD, _DIM)

    out5 = pl.kernel(
        _gather_body,
        out_type=jax.ShapeDtypeStruct((_L, 8, _NW, 8, _BG), jnp.float32),
        mesh=mesh,
        scratch_types=[
            pltpu.VMEM((_L, _BG), jnp.int32),
            pltpu.VMEM((_L, _DIM), jnp.float32),
            pltpu.VMEM((_BG, _DIM), jnp.float32),
            pltpu.VMEM((_BG, _DIM), jnp.float32),
            pltpu.VMEM((8, 8, _BG), jnp.float32),
            pltpu.VMEM((8, 8, _BG), jnp.float32),
            pltpu.SemaphoreType.DMA,
            pltpu.SemaphoreType.DMA,
            pltpu.SemaphoreType.DMA,
            pltpu.SemaphoreType.DMA,
        ],
        compiler_params=pltpu.CompilerParams(
            use_tc_tiling_on_sc=False, needs_layout_passes=False
        ),
    )(xt, tab_lin, pe)
    return out5.transpose(2, 4, 0, 1, 3).reshape(_B, _L, _DIM)


# speedup vs baseline: 1.8615x; 1.7150x over previous
import jax
import jax.numpy as jnp
from jax import lax
from jax.experimental import pallas as pl
from jax.experimental.pallas import tpu as pltpu
from jax.experimental.pallas import tpu_sc as plsc

_DIM = 64
_BASE = 10000.0

_NC = 2
_NS = 16
_NW = _NC * _NS

_B = 4096
_L = 200
_BG = _B // _NW
_NPAIR = _L // 2

_V = 1000000
_NT = (_V + 127) // 128
_VPAD = _NT * 128
_T2ROWS = _VPAD // 2
_NT_LO = _NT // _NW
_NT_EXTRA = _NT - _NT_LO * _NW


def _make_pe():
    pos = jnp.arange(_L, dtype=jnp.float32)[:, None]
    div = jnp.exp(
        jnp.arange(0, _DIM, 2, dtype=jnp.float32) * (-jnp.log(_BASE) / _DIM)
    )
    pe = jnp.zeros((_L, _DIM), dtype=jnp.float32)
    pe = pe.at[:, 0::2].set(jnp.sin(pos * div))
    pe = pe.at[:, 1::2].set(jnp.cos(pos * div))
    return pe


def _relayout_body(tabt_hbm, tail2_hbm, t2_hbm, sbufa, sbufb, obufa, obufb,
                   gsema, gsemb, osema, osemb):
    c = lax.axis_index("c")
    s = lax.axis_index("s")
    wid = s * _NC + c

    lo = jnp.minimum(wid, _NT_EXTRA)
    start = wid * _NT_LO + lo
    cnt = jnp.where(wid < _NT_EXTRA, _NT_LO + 1, _NT_LO)
    cnt = jnp.where(wid == _NW - 1, cnt - 1, cnt)

    iota = lax.iota(jnp.int32, 16)
    qv = [lax.shift_right_logical(iota, 1) + 8 * l for l in range(8)]
    cb = lax.shift_left(lax.bitwise_and(iota, 1), 6)

    def fetch(v, buf, sem):
        return pltpu.async_copy(tabt_hbm.at[:, pl.ds(v * 128, 128)], buf, sem)

    def transpose(sbuf, obuf):
        def frow(f, _):
            cv = cb + f
            for l in range(8):
                v = sbuf[f, pl.ds(l * 16, 16)]
                plsc.store_scatter(obuf, [qv[l], cv], v)
            return 0

        lax.fori_loop(0, 64, frow, 0)

    def put(v, obuf, sem):
        return pltpu.async_copy(t2_hbm.at[pl.ds(v * 64, 64), :], obuf, sem)

    fetch(start, sbufa, gsema)

    def step(i, _):
        va = start + 2 * i
        in_b = va + 1 < start + cnt

        @pl.when(in_b)
        def _():
            fetch(va + 1, sbufb, gsemb)

        pltpu.make_async_copy(
            tabt_hbm.at[:, pl.ds(va * 128, 128)], sbufa, gsema
        ).wait()
        transpose(sbufa, obufa)
        pltpu.sync_copy(
            obufa.at[:, pl.ds(0, 128)], t2_hbm.at[pl.ds(va * 64, 64), :]
        )

        @pl.when(va + 2 < start + cnt)
        def _():
            fetch(va + 2, sbufa, gsema)

        @pl.when(in_b)
        def _():
            pltpu.make_async_copy(
                tabt_hbm.at[:, pl.ds((va + 1) * 128, 128)], sbufb, gsemb
            ).wait()
            transpose(sbufb, obufb)
            pltpu.sync_copy(
                obufb.at[:, pl.ds(0, 128)],
                t2_hbm.at[pl.ds((va + 1) * 64, 64), :],
            )

        return 0

    nsteps = lax.div(cnt + 1, 2)
    lax.fori_loop(0, nsteps, step, 0)

    @pl.when(wid == _NW - 1)
    def _():
        pltpu.sync_copy(tail2_hbm, sbufa.at[pl.ds(0, 32), :])
        pltpu.sync_copy(
            sbufa.at[pl.ds(0, 32), :],
            t2_hbm.at[pl.ds((_NT - 1) * 64, 32), :],
        )


_TAIL0 = (_NT - 1) * 128


def _gather_body(
    xt_hbm, tab_hbm, pe_hbm, out_hbm,
    idx_v, pe_v, gbufa, gbufb, obufa, obufb,
    gsema, gsemb, osema, osemb,
):
    c = lax.axis_index("c")
    s = lax.axis_index("s")
    wid = s * _NC + c

    pltpu.sync_copy(xt_hbm.at[:, pl.ds(wid * _BG, _BG)], idx_v)
    pltpu.sync_copy(pe_hbm, pe_v)

    iota = lax.iota(jnp.int32, 16)
    fgv = [lax.shift_right_logical(iota + 16 * k, 3) for k in range(4)]
    f8v = [lax.bitwise_and(iota + 16 * k, 7) for k in range(4)]

    def gather(j, buf, sem):
        return pltpu.async_copy(tab_hbm.at[idx_v.at[j]], buf, sem)

    def transpose_add(j, gbuf, obuf):
        pe_row = [pe_v[j, pl.ds(k * 16, 16)] for k in range(_DIM // 16)]

        def brow(b, _):
            bv = lax.full((16,), 0, jnp.int32) + b
            for k in range(_DIM // 16):
                v = gbuf[b, pl.ds(k * 16, 16)] + pe_row[k]
                plsc.store_scatter(obuf, [fgv[k], f8v[k], bv], v)
            return 0

        lax.fori_loop(0, _BG, brow, 0)

    def put(j, obuf, sem):
        return pltpu.async_copy(
            obuf.at[:, :, pl.ds(0, _BG)], out_hbm.at[j, :, wid, :, :], sem
        )

    gather(0, gbufa, gsema)

    def pair(m, _):
        ja = 2 * m
        gather(ja + 1, gbufb, gsemb)
        pltpu.make_async_copy(tab_hbm.at[idx_v.at[ja]], gbufa, gsema).wait()
        transpose_add(ja, gbufa, obufa)

        @pl.when(m > 0)
        def _():
            pltpu.make_async_copy(
                obufb.at[:, :, pl.ds(0, _BG)], out_hbm.at[ja - 1, :, wid, :, :], osemb
            ).wait()

        out_a = put(ja, obufa, osema)

        @pl.when(m < _NPAIR - 1)
        def _():
            gather(ja + 2, gbufa, gsema)

        pltpu.make_async_copy(
            tab_hbm.at[idx_v.at[ja + 1]], gbufb, gsemb
        ).wait()
        transpose_add(ja + 1, gbufb, obufb)
        out_a.wait()
        put(ja + 1, obufb, osemb)
        return 0

    lax.fori_loop(0, _NPAIR, pair, 0)
    pltpu.make_async_copy(
        obufb.at[:, :, pl.ds(0, _BG)], out_hbm.at[_L - 1, :, wid, :, :], osemb
    ).wait()


@jax.jit
def kernel(x, table):
    pe = _make_pe()
    xt = x.T.astype(jnp.int32)
    tabt = table.T

    mesh = plsc.VectorSubcoreMesh(core_axis_name="c", subcore_axis_name="s")

    t2 = pl.kernel(
        _relayout_body,
        out_type=jax.ShapeDtypeStruct((_T2ROWS, 128), jnp.float32),
        mesh=mesh,
        scratch_types=[
            pltpu.VMEM((_DIM, 128), jnp.float32),
            pltpu.VMEM((_DIM, 128), jnp.float32),
            pltpu.VMEM((_DIM, 129), jnp.float32),
            pltpu.VMEM((_DIM, 129), jnp.float32),
            pltpu.SemaphoreType.DMA,
            pltpu.SemaphoreType.DMA,
            pltpu.SemaphoreType.DMA,
            pltpu.SemaphoreType.DMA,
        ],
        compiler_params=pltpu.CompilerParams(
            use_tc_tiling_on_sc=True, needs_layout_passes=False
        ),
    )(tabt, table[_TAIL0:, :].reshape(32, 128))

    tab_lin = t2.reshape(_VPAD, _DIM)

    out5 = pl.kernel(
        _gather_body,
        out_type=jax.ShapeDtypeStruct((_L, 8, _NW, 8, _BG), jnp.float32),
        mesh=mesh,
        scratch_types=[
            pltpu.VMEM((_L, _BG), jnp.int32),
            pltpu.VMEM((_L, _DIM), jnp.float32),
            pltpu.VMEM((_BG, _DIM), jnp.float32),
            pltpu.VMEM((_BG, _DIM), jnp.float32),
            pltpu.VMEM((8, 8, _BG + 1), jnp.float32),
            pltpu.VMEM((8, 8, _BG + 1), jnp.float32),
            pltpu.SemaphoreType.DMA,
            pltpu.SemaphoreType.DMA,
            pltpu.SemaphoreType.DMA,
            pltpu.SemaphoreType.DMA,
        ],
        compiler_params=pltpu.CompilerParams(
            use_tc_tiling_on_sc=False, needs_layout_passes=False
        ),
    )(xt, tab_lin, pe)
    return out5.transpose(2, 4, 0, 1, 3).reshape(_B, _L, _DIM)


# speedup vs baseline: 1.9411x; 1.0428x over previous
import jax
import jax.numpy as jnp
from jax import lax
from jax.experimental import pallas as pl
from jax.experimental.pallas import tpu as pltpu
from jax.experimental.pallas import tpu_sc as plsc

_DIM = 64
_BASE = 10000.0

_NC = 2
_NS = 16
_NW = _NC * _NS

_B = 4096
_L = 200
_BG = _B // _NW
_NPAIR = _L // 2

_V = 1000000
_NT = (_V + 127) // 128
_VPAD = _NT * 128
_T2ROWS = _VPAD // 2
_NT_LO = _NT // _NW
_NT_EXTRA = _NT - _NT_LO * _NW


def _make_pe():
    pos = jnp.arange(_L, dtype=jnp.float32)[:, None]
    div = jnp.exp(
        jnp.arange(0, _DIM, 2, dtype=jnp.float32) * (-jnp.log(_BASE) / _DIM)
    )
    pe = jnp.zeros((_L, _DIM), dtype=jnp.float32)
    pe = pe.at[:, 0::2].set(jnp.sin(pos * div))
    pe = pe.at[:, 1::2].set(jnp.cos(pos * div))
    return pe


def _relayout_body(tabt_hbm, tail2_hbm, t2_hbm, sbufa, sbufb, obufa, obufb,
                   gsema, gsemb, osema, osemb):
    c = lax.axis_index("c")
    s = lax.axis_index("s")
    wid = s * _NC + c

    lo = jnp.minimum(wid, _NT_EXTRA)
    start = wid * _NT_LO + lo
    cnt = jnp.where(wid < _NT_EXTRA, _NT_LO + 1, _NT_LO)
    cnt = jnp.where(wid == _NW - 1, cnt - 1, cnt)

    iota = lax.iota(jnp.int32, 16)
    qv = [lax.shift_right_logical(iota, 1) + 8 * l for l in range(8)]
    cb = lax.shift_left(lax.bitwise_and(iota, 1), 6)

    def fetch(v, buf, sem):
        return pltpu.async_copy(tabt_hbm.at[:, pl.ds(v * 128, 128)], buf, sem)

    def transpose(sbuf, obuf):
        def frow(f, _):
            cv = cb + f
            for l in range(8):
                v = sbuf[f, pl.ds(l * 16, 16)]
                plsc.store_scatter(obuf, [qv[l], cv], v)
            return 0

        lax.fori_loop(0, 64, frow, 0)

    def put(v, obuf, sem):
        return pltpu.async_copy(
            obuf.at[:, pl.ds(0, 128)], t2_hbm.at[pl.ds(v * 64, 64), :], sem
        )

    def wait_put(v, obuf, sem):
        pltpu.make_async_copy(
            obuf.at[:, pl.ds(0, 128)], t2_hbm.at[pl.ds(v * 64, 64), :], sem
        ).wait()

    fetch(start, sbufa, gsema)

    def step(i, _):
        va = start + 2 * i
        in_b = va + 1 < start + cnt

        @pl.when(in_b)
        def _():
            fetch(va + 1, sbufb, gsemb)

        pltpu.make_async_copy(
            tabt_hbm.at[:, pl.ds(va * 128, 128)], sbufa, gsema
        ).wait()

        @pl.when(i > 0)
        def _():
            wait_put(va, obufa, osema)

        transpose(sbufa, obufa)
        put(va, obufa, osema)

        @pl.when(va + 2 < start + cnt)
        def _():
            fetch(va + 2, sbufa, gsema)

        @pl.when(in_b)
        def _():
            pltpu.make_async_copy(
                tabt_hbm.at[:, pl.ds((va + 1) * 128, 128)], sbufb, gsemb
            ).wait()

            @pl.when(i > 0)
            def _():
                wait_put(va, obufb, osemb)

            transpose(sbufb, obufb)
            put(va + 1, obufb, osemb)

        return 0

    nsteps = lax.div(cnt + 1, 2)
    lax.fori_loop(0, nsteps, step, 0)

    wait_put(start, obufa, osema)

    @pl.when(cnt > 1)
    def _():
        wait_put(start, obufb, osemb)

    @pl.when(wid == _NW - 1)
    def _():
        pltpu.sync_copy(tail2_hbm, sbufa.at[pl.ds(0, 32), :])
        pltpu.sync_copy(
            sbufa.at[pl.ds(0, 32), :],
            t2_hbm.at[pl.ds((_NT - 1) * 64, 32), :],
        )


_TAIL0 = (_NT - 1) * 128


def _gather_body(
    xt_hbm, tab_hbm, pe_hbm, out_hbm,
    idx_v, pe_v, gbufa, gbufb, obufa, obufb,
    gsema, gsemb, osema, osemb,
):
    c = lax.axis_index("c")
    s = lax.axis_index("s")
    wid = s * _NC + c

    pltpu.sync_copy(xt_hbm.at[:, pl.ds(wid * _BG, _BG)], idx_v)
    pltpu.sync_copy(pe_hbm, pe_v)

    iota = lax.iota(jnp.int32, 16)
    fgv = [lax.shift_right_logical(iota + 16 * k, 3) for k in range(4)]
    f8v = [lax.bitwise_and(iota + 16 * k, 7) for k in range(4)]

    def gather(j, buf, sem):
        return pltpu.async_copy(tab_hbm.at[idx_v.at[j]], buf, sem)

    def transpose_add(j, gbuf, obuf):
        pe_row = [pe_v[j, pl.ds(k * 16, 16)] for k in range(_DIM // 16)]

        def brow(b, _):
            bv = lax.full((16,), 0, jnp.int32) + b
            for k in range(_DIM // 16):
                v = gbuf[b, pl.ds(k * 16, 16)] + pe_row[k]
                plsc.store_scatter(obuf, [fgv[k], f8v[k], bv], v)
            return 0

        lax.fori_loop(0, _BG, brow, 0)

    def put(j, obuf, sem):
        return pltpu.async_copy(
            obuf.at[:, :, pl.ds(0, _BG)], out_hbm.at[j, :, wid, :, :], sem
        )

    gather(0, gbufa, gsema)

    def pair(m, _):
        ja = 2 * m
        gather(ja + 1, gbufb, gsemb)
        pltpu.make_async_copy(tab_hbm.at[idx_v.at[ja]], gbufa, gsema).wait()
        transpose_add(ja, gbufa, obufa)

        @pl.when(m > 0)
        def _():
            pltpu.make_async_copy(
                obufb.at[:, :, pl.ds(0, _BG)], out_hbm.at[ja - 1, :, wid, :, :], osemb
            ).wait()

        out_a = put(ja, obufa, osema)

        @pl.when(m < _NPAIR - 1)
        def _():
            gather(ja + 2, gbufa, gsema)

        pltpu.make_async_copy(
            tab_hbm.at[idx_v.at[ja + 1]], gbufb, gsemb
        ).wait()
        transpose_add(ja + 1, gbufb, obufb)
        out_a.wait()
        put(ja + 1, obufb, osemb)
        return 0

    lax.fori_loop(0, _NPAIR, pair, 0)
    pltpu.make_async_copy(
        obufb.at[:, :, pl.ds(0, _BG)], out_hbm.at[_L - 1, :, wid, :, :], osemb
    ).wait()


@jax.jit
def kernel(x, table):
    pe = _make_pe()
    xt = x.T.astype(jnp.int32)
    tabt = table.T

    mesh = plsc.VectorSubcoreMesh(core_axis_name="c", subcore_axis_name="s")

    t2 = pl.kernel(
        _relayout_body,
        out_type=jax.ShapeDtypeStruct((_T2ROWS, 128), jnp.float32),
        mesh=mesh,
        scratch_types=[
            pltpu.VMEM((_DIM, 128), jnp.float32),
            pltpu.VMEM((_DIM, 128), jnp.float32),
            pltpu.VMEM((_DIM, 129), jnp.float32),
            pltpu.VMEM((_DIM, 129), jnp.float32),
            pltpu.SemaphoreType.DMA,
            pltpu.SemaphoreType.DMA,
            pltpu.SemaphoreType.DMA,
            pltpu.SemaphoreType.DMA,
        ],
        compiler_params=pltpu.CompilerParams(
            use_tc_tiling_on_sc=True, needs_layout_passes=False
        ),
    )(tabt, table[_TAIL0:, :].reshape(32, 128))

    tab_lin = t2.reshape(_VPAD, _DIM)

    out5 = pl.kernel(
        _gather_body,
        out_type=jax.ShapeDtypeStruct((_L, 8, _NW, 8, _BG), jnp.float32),
        mesh=mesh,
        scratch_types=[
            pltpu.VMEM((_L, _BG), jnp.int32),
            pltpu.VMEM((_L, _DIM), jnp.float32),
            pltpu.VMEM((_BG, _DIM), jnp.float32),
            pltpu.VMEM((_BG, _DIM), jnp.float32),
            pltpu.VMEM((8, 8, _BG + 1), jnp.float32),
            pltpu.VMEM((8, 8, _BG + 1), jnp.float32),
            pltpu.SemaphoreType.DMA,
            pltpu.SemaphoreType.DMA,
            pltpu.SemaphoreType.DMA,
            pltpu.SemaphoreType.DMA,
        ],
        compiler_params=pltpu.CompilerParams(
            use_tc_tiling_on_sc=False, needs_layout_passes=False
        ),
    )(xt, tab_lin, pe)
    return out5.transpose(2, 4, 0, 1, 3).reshape(_B, _L, _DIM)
